# baseline (speedup 1.0000x reference)
import functools

import jax
import jax.numpy as jnp
from jax import lax
from jax.experimental import pallas as pl
from jax.experimental.pallas import tpu as pltpu
from jax.experimental.pallas import tpu_sc as plsc

N_SITES = 20
N_STATES = 2 ** N_SITES
BATCH = 16384

NUM_CORES = 2
NUM_SUBCORES = 16
LANES = 16
NUM_WORKERS = NUM_CORES * NUM_SUBCORES
B_PER_W = BATCH // NUM_WORKERS
CHUNK = 128
N_CHUNKS = B_PER_W // CHUNK
N_GROUPS = B_PER_W // LANES


def _sc_body(xt_hbm, table_hbm, out_hbm, x_v, idx_v, out_v, sem):
    wid = lax.axis_index("s") * NUM_CORES + lax.axis_index("c")
    base = wid * B_PER_W

    pltpu.sync_copy(xt_hbm.at[:, pl.ds(base, B_PER_W)], x_v)

    def pack_group(g, _):
        off = g * LANES
        num = x_v[0, pl.ds(off, LANES)]
        for site in range(1, N_SITES):
            num = num * 2 + x_v[site, pl.ds(off, LANES)]
        idx_v[pl.ds(off, LANES)] = num
        return _

    lax.fori_loop(0, N_GROUPS, pack_group, None)

    copies = []
    for j in range(N_CHUNKS):
        sl = pl.ds(j * CHUNK, CHUNK)
        copies.append(
            pltpu.async_copy(table_hbm.at[idx_v.at[sl]], out_v.at[sl], sem))
    for c in copies:
        c.wait()

    pltpu.sync_copy(out_v, out_hbm.at[pl.ds(base, B_PER_W)])


@jax.jit
def _sc_lookup(xt, logstate):
    mesh = plsc.VectorSubcoreMesh(core_axis_name="c", subcore_axis_name="s")
    run = pl.kernel(
        _sc_body,
        mesh=mesh,
        out_type=jax.ShapeDtypeStruct((BATCH,), jnp.float32),
        scratch_types=[
            pltpu.VMEM((N_SITES, B_PER_W), jnp.int32),
            pltpu.VMEM((B_PER_W,), jnp.int32),
            pltpu.VMEM((B_PER_W,), jnp.float32),
            pltpu.SemaphoreType.DMA,
        ],
    )
    return run(xt, logstate)


def kernel(x_in, logstate):
    xt = x_in.T.astype(jnp.int32)
    return _sc_lookup(xt, logstate)
